# Initial kernel scaffold; baseline (speedup 1.0000x reference)
import functools

import jax
import jax.numpy as jnp
from jax import lax
from jax.experimental import pallas as pl
from jax.experimental.pallas import tpu as pltpu
from jax.experimental.pallas import tpu_sc as plsc

_NC = 2
_NS = 16
_L = 16


def _build(B, D, RD, BQ):
    A = RD // D
    W = D // BQ
    NW = _NC * _NS
    per_w = B // NW
    C = 64
    n_sub = per_w // C

    mesh = plsc.VectorSubcoreMesh(core_axis_name="c", subcore_axis_name="s")

    scratch = [
        pltpu.VMEM((C,), jnp.int32),
    ] + [pltpu.VMEM((C,), jnp.int32) for _ in range(A)] + [
        pltpu.VMEM((C, RD), jnp.float32),
    ] + [pltpu.VMEM((C, D), jnp.float32) for _ in range(A)] + [
        pltpu.VMEM((C, BQ), jnp.float32),
        pltpu.VMEM((C,), jnp.float32),
        pltpu.SemaphoreType.DMA,
    ]

    @functools.partial(
        pl.kernel,
        out_type=jax.ShapeDtypeStruct((B,), jnp.float32),
        mesh=mesh,
        scratch_types=scratch,
    )
    def k(r_idx_h, e1_h, e2_h, e3_h, e4_h, e5_h, e6_h, E_h, R_h, Rb_h,
          out_h, r_i, i0, i1, i2, i3, i4, i5, r_v, v0, v1, v2, v3, v4, v5,
          rb_v, out_v, sem):
        e_idx_hs = (e1_h, e2_h, e3_h, e4_h, e5_h, e6_h)
        e_is = (i0, i1, i2, i3, i4, i5)
        e_vs = (v0, v1, v2, v3, v4, v5)

        wid = lax.axis_index("s") * _NC + lax.axis_index("c")
        base = wid * per_w
        lane = lax.iota(jnp.int32, _L)
        lane0 = lane == 0

        def sub_body(sub, carry):
            off = base + sub * C
            pltpu.sync_copy(r_idx_h.at[pl.ds(off, C)], r_i)
            for h, iv in zip(e_idx_hs, e_is):
                pltpu.sync_copy(h.at[pl.ds(off, C)], iv)
            descs = [pltpu.async_copy(R_h.at[r_i], r_v, sem),
                     pltpu.async_copy(Rb_h.at[r_i], rb_v, sem)]
            for iv, ev in zip(e_is, e_vs):
                descs.append(pltpu.async_copy(E_h.at[iv], ev, sem))
            for d in descs:
                d.wait()

            def item(i, c):
                inner = rb_v[i, :]
                for a in range(A):
                    ev = e_vs[a]
                    for w in range(W):
                        inner = inner + (r_v[i, pl.ds(a * D + w * BQ, BQ)]
                                         * ev[i, pl.ds(w * BQ, BQ)])
                inner = jnp.maximum(inner, 0.0)
                s = jnp.sum(inner)
                plsc.store_scatter(
                    out_v, [jnp.full((_L,), i, jnp.int32)],
                    jnp.full((_L,), s, jnp.float32), mask=lane0)
                return c

            lax.fori_loop(0, C, item, 0)
            pltpu.sync_copy(out_v, out_h.at[pl.ds(off, C)])
            return carry

        lax.fori_loop(0, n_sub, sub_body, 0)

    return k


def kernel(r_idx, e1_idx, e2_idx, e3_idx, e4_idx, e5_idx, e6_idx, E, R,
           R_bias):
    B = r_idx.shape[0]
    D = E.shape[1]
    RD = R.shape[1]
    BQ = R_bias.shape[1]
    k = _build(B, D, RD, BQ)
    to_i32 = lambda x: x.astype(jnp.int32)
    return k(to_i32(r_idx), to_i32(e1_idx), to_i32(e2_idx), to_i32(e3_idx),
             to_i32(e4_idx), to_i32(e5_idx), to_i32(e6_idx), E, R, R_bias)


# speedup vs baseline: 2.9997x; 2.9997x over previous
import functools

import jax
import jax.numpy as jnp
from jax import lax
from jax.experimental import pallas as pl
from jax.experimental.pallas import tpu as pltpu
from jax.experimental.pallas import tpu_sc as plsc

_NC = 2
_NS = 16
_L = 16


def _build(B, D, RD, BQ):
    A = (RD - D) // D
    W = D // BQ
    NW = _NC * _NS
    per_w = B // NW
    C = 64
    n_sub = per_w // C

    mesh = plsc.VectorSubcoreMesh(core_axis_name="c", subcore_axis_name="s")

    scratch = [
        pltpu.VMEM((C,), jnp.int32),
    ] + [pltpu.VMEM((C,), jnp.int32) for _ in range(A)] + [
        pltpu.VMEM((C, RD), jnp.float32),
    ] + [pltpu.VMEM((C, D), jnp.float32) for _ in range(A)] + [
        pltpu.VMEM((C,), jnp.float32),
        pltpu.SemaphoreType.DMA,
    ]

    @functools.partial(
        pl.kernel,
        out_type=jax.ShapeDtypeStruct((B,), jnp.float32),
        mesh=mesh,
        scratch_types=scratch,
    )
    def k(r_idx_h, e1_h, e2_h, e3_h, e4_h, e5_h, e6_h, E_h, R_h,
          out_h, r_i, i0, i1, i2, i3, i4, i5, r_v, v0, v1, v2, v3, v4, v5,
          out_v, sem):
        e_idx_hs = (e1_h, e2_h, e3_h, e4_h, e5_h, e6_h)
        e_is = (i0, i1, i2, i3, i4, i5)
        e_vs = (v0, v1, v2, v3, v4, v5)

        wid = lax.axis_index("s") * _NC + lax.axis_index("c")
        base = wid * per_w
        lane = lax.iota(jnp.int32, _L)
        lane0 = lane == 0

        def sub_body(sub, carry):
            off = base + sub * C
            pltpu.sync_copy(r_idx_h.at[pl.ds(off, C)], r_i)
            for h, iv in zip(e_idx_hs, e_is):
                pltpu.sync_copy(h.at[pl.ds(off, C)], iv)
            descs = [pltpu.async_copy(R_h.at[r_i], r_v, sem)]
            for iv, ev in zip(e_is, e_vs):
                descs.append(pltpu.async_copy(E_h.at[iv], ev, sem))
            for d in descs:
                d.wait()

            def group(g, carry_g):
                def item(t, acc):
                    i = g * _L + t
                    inner = r_v[i, pl.ds(A * D, BQ)]
                    for a in range(A):
                        ev = e_vs[a]
                        for w in range(W):
                            inner = inner + (
                                r_v[i, pl.ds(a * D + w * BQ, BQ)]
                                * ev[i, pl.ds(w * BQ, BQ)])
                    inner = jnp.maximum(inner, 0.0)
                    for step in (8, 4, 2, 1):
                        inner = inner + inner.at[lane ^ step].get(
                            mode="promise_in_bounds")
                    return jnp.where(lane == t, inner, acc)

                acc = lax.fori_loop(0, _L, item,
                                    jnp.zeros((_L,), jnp.float32))
                out_v[pl.ds(g * _L, _L)] = acc
                return carry_g

            lax.fori_loop(0, C // _L, group, 0)
            pltpu.sync_copy(out_v, out_h.at[pl.ds(off, C)])
            return carry

        lax.fori_loop(0, n_sub, sub_body, 0)

    return k


def kernel(r_idx, e1_idx, e2_idx, e3_idx, e4_idx, e5_idx, e6_idx, E, R,
           R_bias):
    B = r_idx.shape[0]
    D = E.shape[1]
    BQ = R_bias.shape[1]
    nrel = R.shape[0]
    rb_pad = jnp.zeros((nrel, D), R_bias.dtype).at[:, :BQ].set(R_bias)
    Rf = jnp.concatenate([R, rb_pad], axis=1)
    k = _build(B, D, Rf.shape[1], BQ)
    to_i32 = lambda x: x.astype(jnp.int32)
    return k(to_i32(r_idx), to_i32(e1_idx), to_i32(e2_idx), to_i32(e3_idx),
             to_i32(e4_idx), to_i32(e5_idx), to_i32(e6_idx), E, Rf)


# speedup vs baseline: 4.1165x; 1.3723x over previous
import functools

import jax
import jax.numpy as jnp
from jax import lax
from jax.experimental import pallas as pl
from jax.experimental.pallas import tpu as pltpu
from jax.experimental.pallas import tpu_sc as plsc

_NC = 2
_NS = 16
_L = 16


def _build(B, D, RD, BQ):
    A = (RD - D) // D
    W = D // BQ
    NW = _NC * _NS
    per_w = B // NW
    C = 32
    n_sub = per_w // C

    mesh = plsc.VectorSubcoreMesh(core_axis_name="c", subcore_axis_name="s")

    scratch = (
        [pltpu.VMEM((n_sub, C), jnp.int32) for _ in range(1 + A)]
        + [pltpu.VMEM((C, RD), jnp.float32) for _ in range(2)]
        + [pltpu.VMEM((C, D), jnp.float32) for _ in range(2 * A)]
        + [pltpu.VMEM((per_w,), jnp.float32)]
        + [pltpu.SemaphoreType.DMA, pltpu.SemaphoreType.DMA]
    )

    @functools.partial(
        pl.kernel,
        out_type=jax.ShapeDtypeStruct((B,), jnp.float32),
        mesh=mesh,
        scratch_types=scratch,
    )
    def k(r_idx_h, e1_h, e2_h, e3_h, e4_h, e5_h, e6_h, E_h, R_h,
          out_h, ri, i0, i1, i2, i3, i4, i5, rv0, rv1,
          v00, v01, v02, v03, v04, v05, v10, v11, v12, v13, v14, v15,
          out_b, sem0, sem1):
        idx_hs = (r_idx_h, e1_h, e2_h, e3_h, e4_h, e5_h, e6_h)
        idx_vs = (ri, i0, i1, i2, i3, i4, i5)
        bufs = (
            (rv0, (v00, v01, v02, v03, v04, v05), sem0),
            (rv1, (v10, v11, v12, v13, v14, v15), sem1),
        )

        wid = lax.axis_index("s") * _NC + lax.axis_index("c")
        base = wid * per_w
        lane = lax.iota(jnp.int32, _L)

        for h, iv in zip(idx_hs, idx_vs):
            pltpu.sync_copy(h.at[wid], iv)

        def fire(sub, bset):
            rv, evs, sem = bset
            pltpu.async_copy(R_h.at[ri.at[sub]], rv, sem)
            for a in range(A):
                pltpu.async_copy(E_h.at[idx_vs[1 + a].at[sub]], evs[a], sem)

        def drain(sub, bset):
            rv, evs, sem = bset
            pltpu.make_async_copy(R_h.at[ri.at[sub]], rv, sem).wait()
            for a in range(A):
                pltpu.make_async_copy(
                    E_h.at[idx_vs[1 + a].at[sub]], evs[a], sem).wait()

        def compute(sub, bset):
            rv, evs, _ = bset

            def group(g, carry_g):
                def item(t, acc):
                    i = g * _L + t
                    parts = [rv[i, pl.ds(A * D, BQ)],
                             jnp.zeros((_L,), jnp.float32),
                             jnp.zeros((_L,), jnp.float32),
                             jnp.zeros((_L,), jnp.float32)]
                    n = 0
                    for a in range(A):
                        ev = evs[a]
                        for w in range(W):
                            parts[n % 4] = parts[n % 4] + (
                                rv[i, pl.ds(a * D + w * BQ, BQ)]
                                * ev[i, pl.ds(w * BQ, BQ)])
                            n += 1
                    inner = (parts[0] + parts[1]) + (parts[2] + parts[3])
                    inner = jnp.maximum(inner, 0.0)
                    for step in (8, 4, 2, 1):
                        inner = inner + inner.at[lane ^ step].get(
                            mode="promise_in_bounds")
                    return jnp.where(lane == t, inner, acc)

                acc = lax.fori_loop(0, _L, item,
                                    jnp.zeros((_L,), jnp.float32))
                out_b[pl.ds(sub * C + g * _L, _L)] = acc
                return carry_g

            lax.fori_loop(0, C // _L, group, 0)

        fire(0, bufs[0])

        def pair(it, carry):
            sub = 2 * it
            fire(sub + 1, bufs[1])
            drain(sub, bufs[0])
            compute(sub, bufs[0])
            fire(sub + 2, bufs[0])
            drain(sub + 1, bufs[1])
            compute(sub + 1, bufs[1])
            return carry

        lax.fori_loop(0, n_sub // 2 - 1, pair, 0)

        s_last = n_sub - 2
        fire(s_last + 1, bufs[1])
        drain(s_last, bufs[0])
        compute(s_last, bufs[0])
        drain(s_last + 1, bufs[1])
        compute(s_last + 1, bufs[1])

        pltpu.sync_copy(out_b, out_h.at[pl.ds(base, per_w)])

    return k


def kernel(r_idx, e1_idx, e2_idx, e3_idx, e4_idx, e5_idx, e6_idx, E, R,
           R_bias):
    B = r_idx.shape[0]
    D = E.shape[1]
    BQ = R_bias.shape[1]
    nrel = R.shape[0]
    rb_pad = jnp.zeros((nrel, D), R_bias.dtype).at[:, :BQ].set(R_bias)
    Rf = jnp.concatenate([R, rb_pad], axis=1)
    k = _build(B, D, Rf.shape[1], BQ)
    NW = _NC * _NS
    per_w = B // NW
    C = 32
    shp = (NW, per_w // C, C)
    to_idx = lambda x: x.astype(jnp.int32).reshape(shp)
    return k(to_idx(r_idx), to_idx(e1_idx), to_idx(e2_idx), to_idx(e3_idx),
             to_idx(e4_idx), to_idx(e5_idx), to_idx(e6_idx), E, Rf)


# speedup vs baseline: 5.1487x; 1.2508x over previous
import functools

import jax
import jax.numpy as jnp
from jax import lax
from jax.experimental import pallas as pl
from jax.experimental.pallas import tpu as pltpu
from jax.experimental.pallas import tpu_sc as plsc

_NC = 2
_NS = 16
_L = 16


def _build(B, D, RD, BQ):
    A = (RD - D) // D
    W = D // BQ
    NW = _NC * _NS
    per_w = B // NW
    C = 32
    n_sub = per_w // C

    mesh = plsc.VectorSubcoreMesh(core_axis_name="c", subcore_axis_name="s")

    scratch = (
        [pltpu.VMEM((per_w,), jnp.int32) for _ in range(1 + A)]
        + [pltpu.VMEM((C, RD), jnp.float32) for _ in range(2)]
        + [pltpu.VMEM((C, D), jnp.float32) for _ in range(2 * A)]
        + [pltpu.VMEM((per_w,), jnp.float32)]
        + [pltpu.SemaphoreType.DMA, pltpu.SemaphoreType.DMA,
           pltpu.SemaphoreType.DMA]
    )

    @functools.partial(
        pl.kernel,
        out_type=jax.ShapeDtypeStruct((B,), jnp.float32),
        mesh=mesh,
        scratch_types=scratch,
    )
    def k(r_idx_h, e1_h, e2_h, e3_h, e4_h, e5_h, e6_h, E_h, R_h,
          out_h, ri, i0, i1, i2, i3, i4, i5, rv0, rv1,
          v00, v01, v02, v03, v04, v05, v10, v11, v12, v13, v14, v15,
          out_b, semi, sem0, sem1):
        idx_hs = (r_idx_h, e1_h, e2_h, e3_h, e4_h, e5_h, e6_h)
        idx_vs = (ri, i0, i1, i2, i3, i4, i5)
        bufs = (
            (rv0, (v00, v01, v02, v03, v04, v05), sem0),
            (rv1, (v10, v11, v12, v13, v14, v15), sem1),
        )

        wid = lax.axis_index("s") * _NC + lax.axis_index("c")
        base = wid * per_w
        lane = lax.iota(jnp.int32, _L)

        stage = [pltpu.async_copy(h.at[pl.ds(base, per_w)], iv, semi)
                 for h, iv in zip(idx_hs, idx_vs)]
        for d in stage:
            d.wait()

        def fire(sub, bset):
            rv, evs, sem = bset
            pltpu.async_copy(R_h.at[ri.at[pl.ds(sub * C, C)]], rv, sem)
            for a in range(A):
                pltpu.async_copy(
                    E_h.at[idx_vs[1 + a].at[pl.ds(sub * C, C)]], evs[a], sem)

        def drain(sub, bset):
            rv, evs, sem = bset
            pltpu.make_async_copy(
                R_h.at[ri.at[pl.ds(sub * C, C)]], rv, sem).wait()
            for a in range(A):
                pltpu.make_async_copy(
                    E_h.at[idx_vs[1 + a].at[pl.ds(sub * C, C)]],
                    evs[a], sem).wait()

        def compute(sub, bset):
            rv, evs, _ = bset

            def group(g, carry_g):
                def item(t, acc):
                    i = g * _L + t
                    parts = [rv[i, pl.ds(A * D, BQ)],
                             jnp.zeros((_L,), jnp.float32)]
                    n = 0
                    for a in range(A):
                        ev = evs[a]
                        for w in range(W):
                            parts[n % 2] = parts[n % 2] + (
                                rv[i, pl.ds(a * D + w * BQ, BQ)]
                                * ev[i, pl.ds(w * BQ, BQ)])
                            n += 1
                    inner = parts[0] + parts[1]
                    inner = jnp.maximum(inner, 0.0)
                    for step in (8, 4, 2, 1):
                        inner = inner + inner.at[lane ^ step].get(
                            mode="promise_in_bounds")
                    return jnp.where(lane == t, inner, acc)

                acc = lax.fori_loop(0, _L, item,
                                    jnp.zeros((_L,), jnp.float32))
                out_b[pl.ds(sub * C + g * _L, _L)] = acc
                return carry_g

            lax.fori_loop(0, C // _L, group, 0)

        fire(0, bufs[0])

        def pair(it, carry):
            sub = 2 * it
            fire(sub + 1, bufs[1])
            drain(sub, bufs[0])
            compute(sub, bufs[0])
            fire(sub + 2, bufs[0])
            drain(sub + 1, bufs[1])
            compute(sub + 1, bufs[1])
            return carry

        lax.fori_loop(0, n_sub // 2 - 1, pair, 0)

        s_last = n_sub - 2
        fire(s_last + 1, bufs[1])
        drain(s_last, bufs[0])
        compute(s_last, bufs[0])
        drain(s_last + 1, bufs[1])
        compute(s_last + 1, bufs[1])

        pltpu.sync_copy(out_b, out_h.at[pl.ds(base, per_w)])

    return k


def kernel(r_idx, e1_idx, e2_idx, e3_idx, e4_idx, e5_idx, e6_idx, E, R,
           R_bias):
    B = r_idx.shape[0]
    D = E.shape[1]
    BQ = R_bias.shape[1]
    nrel = R.shape[0]
    rb_pad = jnp.zeros((nrel, D), R_bias.dtype).at[:, :BQ].set(R_bias)
    Rf = jnp.concatenate([R, rb_pad], axis=1)
    k = _build(B, D, Rf.shape[1], BQ)
    to_i32 = lambda x: x.astype(jnp.int32)
    return k(to_i32(r_idx), to_i32(e1_idx), to_i32(e2_idx), to_i32(e3_idx),
             to_i32(e4_idx), to_i32(e5_idx), to_i32(e6_idx), E, Rf)


# speedup vs baseline: 5.2390x; 1.0175x over previous
import functools

import jax
import jax.numpy as jnp
from jax import lax
from jax.experimental import pallas as pl
from jax.experimental.pallas import tpu as pltpu
from jax.experimental.pallas import tpu_sc as plsc

_NC = 2
_NS = 16
_L = 16


def _build(B, D, RD, BQ, nrel):
    A = RD // D
    W = D // BQ
    NW = _NC * _NS
    per_w = B // NW
    C = 32
    n_sub = per_w // C

    mesh = plsc.VectorSubcoreMesh(core_axis_name="c", subcore_axis_name="s")

    scratch = (
        [pltpu.VMEM((per_w + _L,), jnp.int32)]
        + [pltpu.VMEM((per_w,), jnp.int32) for _ in range(A)]
        + [pltpu.VMEM((C, RD), jnp.float32) for _ in range(2)]
        + [pltpu.VMEM((C, D), jnp.float32) for _ in range(2 * A)]
        + [pltpu.VMEM((nrel * BQ,), jnp.float32)]
        + [pltpu.VMEM((per_w,), jnp.float32)]
        + [pltpu.SemaphoreType.DMA, pltpu.SemaphoreType.DMA,
           pltpu.SemaphoreType.DMA]
    )

    @functools.partial(
        pl.kernel,
        out_type=jax.ShapeDtypeStruct((B,), jnp.float32),
        mesh=mesh,
        scratch_types=scratch,
    )
    def k(r_idx_h, e1_h, e2_h, e3_h, e4_h, e5_h, e6_h, E_h, R_h, Rb_h,
          out_h, ri, i0, i1, i2, i3, i4, i5, rv0, rv1,
          v00, v01, v02, v03, v04, v05, v10, v11, v12, v13, v14, v15,
          rb_full, out_b, semi, sem0, sem1):
        idx_hs = (r_idx_h, e1_h, e2_h, e3_h, e4_h, e5_h, e6_h)
        idx_vs = (ri, i0, i1, i2, i3, i4, i5)
        bufs = (
            (rv0, (v00, v01, v02, v03, v04, v05), sem0),
            (rv1, (v10, v11, v12, v13, v14, v15), sem1),
        )

        wid = lax.axis_index("s") * _NC + lax.axis_index("c")
        base = wid * per_w
        lane = lax.iota(jnp.int32, _L)

        stage = [pltpu.async_copy(h.at[pl.ds(base, per_w)],
                                  iv.at[pl.ds(0, per_w)] if iv is ri else iv,
                                  semi)
                 for h, iv in zip(idx_hs, idx_vs)]
        stage.append(pltpu.async_copy(Rb_h, rb_full, semi))
        for d in stage:
            d.wait()

        def fire(sub, bset):
            rv, evs, sem = bset
            pltpu.async_copy(R_h.at[ri.at[pl.ds(sub * C, C)]], rv, sem)
            for a in range(A):
                pltpu.async_copy(
                    E_h.at[idx_vs[1 + a].at[pl.ds(sub * C, C)]], evs[a], sem)

        def drain(sub, bset):
            rv, evs, sem = bset
            pltpu.make_async_copy(
                R_h.at[ri.at[pl.ds(sub * C, C)]], rv, sem).wait()
            for a in range(A):
                pltpu.make_async_copy(
                    E_h.at[idx_vs[1 + a].at[pl.ds(sub * C, C)]],
                    evs[a], sem).wait()

        def compute(sub, bset):
            rv, evs, _ = bset

            def group(g, carry_g):
                def item(t, acc):
                    i = g * _L + t
                    rid = ri[pl.ds(sub * C + i, _L)][0]
                    parts = [rb_full[pl.ds(rid * BQ, BQ)],
                             jnp.zeros((_L,), jnp.float32)]
                    n = 0
                    for a in range(A):
                        ev = evs[a]
                        for w in range(W):
                            parts[n % 2] = parts[n % 2] + (
                                rv[i, pl.ds(a * D + w * BQ, BQ)]
                                * ev[i, pl.ds(w * BQ, BQ)])
                            n += 1
                    inner = parts[0] + parts[1]
                    inner = jnp.maximum(inner, 0.0)
                    for step in (8, 4, 2, 1):
                        inner = inner + inner.at[lane ^ step].get(
                            mode="promise_in_bounds")
                    return jnp.where(lane == t, inner, acc)

                acc = lax.fori_loop(0, _L, item,
                                    jnp.zeros((_L,), jnp.float32))
                out_b[pl.ds(sub * C + g * _L, _L)] = acc
                return carry_g

            lax.fori_loop(0, C // _L, group, 0)

        fire(0, bufs[0])

        def pair(it, carry):
            sub = 2 * it
            fire(sub + 1, bufs[1])
            drain(sub, bufs[0])
            compute(sub, bufs[0])
            fire(sub + 2, bufs[0])
            drain(sub + 1, bufs[1])
            compute(sub + 1, bufs[1])
            return carry

        lax.fori_loop(0, n_sub // 2 - 1, pair, 0)

        s_last = n_sub - 2
        fire(s_last + 1, bufs[1])
        drain(s_last, bufs[0])
        compute(s_last, bufs[0])
        drain(s_last + 1, bufs[1])
        compute(s_last + 1, bufs[1])

        pltpu.sync_copy(out_b, out_h.at[pl.ds(base, per_w)])

    return k


def kernel(r_idx, e1_idx, e2_idx, e3_idx, e4_idx, e5_idx, e6_idx, E, R,
           R_bias):
    B = r_idx.shape[0]
    D = E.shape[1]
    RD = R.shape[1]
    BQ = R_bias.shape[1]
    nrel = R.shape[0]
    k = _build(B, D, RD, BQ, nrel)
    to_i32 = lambda x: x.astype(jnp.int32)
    return k(to_i32(r_idx), to_i32(e1_idx), to_i32(e2_idx), to_i32(e3_idx),
             to_i32(e4_idx), to_i32(e5_idx), to_i32(e6_idx), E, R,
             R_bias.reshape(-1))


# speedup vs baseline: 5.3213x; 1.0157x over previous
import functools

import jax
import jax.numpy as jnp
from jax import lax
from jax.experimental import pallas as pl
from jax.experimental.pallas import tpu as pltpu
from jax.experimental.pallas import tpu_sc as plsc

_NC = 2
_NS = 16
_L = 16


def _build(B, D, RD, BQ, nrel):
    A = RD // D
    W = D // BQ
    NW = _NC * _NS
    per_w = B // NW
    C = 32
    n_sub = per_w // C

    mesh = plsc.VectorSubcoreMesh(core_axis_name="c", subcore_axis_name="s")

    scratch = (
        [pltpu.VMEM((per_w + _L,), jnp.int32)]
        + [pltpu.VMEM((per_w,), jnp.int32) for _ in range(A)]
        + [pltpu.VMEM((C, RD), jnp.float32) for _ in range(2)]
        + [pltpu.VMEM((C, D), jnp.float32) for _ in range(2 * A)]
        + [pltpu.VMEM((nrel * BQ,), jnp.float32)]
        + [pltpu.VMEM((per_w,), jnp.float32)]
        + [pltpu.SemaphoreType.DMA, pltpu.SemaphoreType.DMA,
           pltpu.SemaphoreType.DMA]
    )

    @functools.partial(
        pl.kernel,
        out_type=jax.ShapeDtypeStruct((B,), jnp.float32),
        mesh=mesh,
        scratch_types=scratch,
    )
    def k(r_idx_h, e1_h, e2_h, e3_h, e4_h, e5_h, e6_h, E_h, R_h, Rb_h,
          out_h, ri, i0, i1, i2, i3, i4, i5, rv0, rv1,
          v00, v01, v02, v03, v04, v05, v10, v11, v12, v13, v14, v15,
          rb_full, out_b, semi, sem0, sem1):
        idx_hs = (r_idx_h, e1_h, e2_h, e3_h, e4_h, e5_h, e6_h)
        idx_vs = (ri, i0, i1, i2, i3, i4, i5)
        bufs = (
            (rv0, (v00, v01, v02, v03, v04, v05), sem0),
            (rv1, (v10, v11, v12, v13, v14, v15), sem1),
        )

        wid = lax.axis_index("s") * _NC + lax.axis_index("c")
        base = wid * per_w
        lane = lax.iota(jnp.int32, _L)

        stage = [pltpu.async_copy(h.at[pl.ds(base, per_w)],
                                  iv.at[pl.ds(0, per_w)] if iv is ri else iv,
                                  semi)
                 for h, iv in zip(idx_hs, idx_vs)]
        stage.append(pltpu.async_copy(Rb_h, rb_full, semi))
        for d in stage:
            d.wait()

        def fire(sub, bset):
            rv, evs, sem = bset
            pltpu.async_copy(R_h.at[ri.at[pl.ds(sub * C, C)]], rv, sem)
            for a in range(A):
                pltpu.async_copy(
                    E_h.at[idx_vs[1 + a].at[pl.ds(sub * C, C)]], evs[a], sem)

        def drain(sub, bset):
            rv, evs, sem = bset
            pltpu.make_async_copy(
                R_h.at[ri.at[pl.ds(sub * C, C)]], rv, sem).wait()
            for a in range(A):
                pltpu.make_async_copy(
                    E_h.at[idx_vs[1 + a].at[pl.ds(sub * C, C)]],
                    evs[a], sem).wait()

        def compute(sub, bset):
            rv, evs, _ = bset

            def group(g, carry_g):
                def item(t, acc):
                    i = g * _L + t
                    rid = ri[pl.ds(sub * C + i, _L)][0]
                    parts = [rb_full[pl.ds(rid * BQ, BQ)],
                             jnp.zeros((_L,), jnp.float32)]
                    n = 0
                    for a in range(A):
                        ev = evs[a]
                        for w in range(W):
                            parts[n % 2] = parts[n % 2] + (
                                rv[i, pl.ds(a * D + w * BQ, BQ)]
                                * ev[i, pl.ds(w * BQ, BQ)])
                            n += 1
                    inner = parts[0] + parts[1]
                    inner = jnp.maximum(inner, 0.0)
                    for step in (8, 4, 2, 1):
                        inner = inner + inner.at[lane ^ step].get(
                            mode="promise_in_bounds")
                    return jnp.where(lane == t, inner, acc)

                acc = lax.fori_loop(0, _L, item,
                                    jnp.zeros((_L,), jnp.float32))
                out_b[pl.ds(sub * C + g * _L, _L)] = acc
                return carry_g

            lax.fori_loop(0, C // _L, group, 0)

        fire(0, bufs[0])

        def pair(it, carry):
            sub = 2 * it
            fire(sub + 1, bufs[1])
            drain(sub, bufs[0])
            compute(sub, bufs[0])
            fire(lax.rem(sub + 2, n_sub), bufs[0])
            drain(sub + 1, bufs[1])
            compute(sub + 1, bufs[1])
            return carry

        lax.fori_loop(0, n_sub // 2, pair, 0)
        drain(0, bufs[0])

        pltpu.sync_copy(out_b, out_h.at[pl.ds(base, per_w)])

    return k


def kernel(r_idx, e1_idx, e2_idx, e3_idx, e4_idx, e5_idx, e6_idx, E, R,
           R_bias):
    B = r_idx.shape[0]
    D = E.shape[1]
    RD = R.shape[1]
    BQ = R_bias.shape[1]
    nrel = R.shape[0]
    k = _build(B, D, RD, BQ, nrel)
    to_i32 = lambda x: x.astype(jnp.int32)
    return k(to_i32(r_idx), to_i32(e1_idx), to_i32(e2_idx), to_i32(e3_idx),
             to_i32(e4_idx), to_i32(e5_idx), to_i32(e6_idx), E, R,
             R_bias.reshape(-1))


# speedup vs baseline: 5.4209x; 1.0187x over previous
import functools

import jax
import jax.numpy as jnp
from jax import lax
from jax.experimental import pallas as pl
from jax.experimental.pallas import tpu as pltpu
from jax.experimental.pallas import tpu_sc as plsc

_NC = 2
_NS = 16
_L = 16


def _build(B, D, RD, BQ, nrel):
    A = RD // D
    W = D // BQ
    NW = _NC * _NS
    per_w = B // NW
    C = 32
    n_sub = per_w // C

    mesh = plsc.VectorSubcoreMesh(core_axis_name="c", subcore_axis_name="s")

    scratch = (
        [pltpu.VMEM((per_w + _L,), jnp.int32)]
        + [pltpu.VMEM((per_w,), jnp.int32) for _ in range(A)]
        + [pltpu.VMEM((2, C, RD), jnp.float32)]
        + [pltpu.VMEM((2, C, D), jnp.float32) for _ in range(A)]
        + [pltpu.VMEM((nrel * BQ,), jnp.float32)]
        + [pltpu.VMEM((per_w,), jnp.float32)]
        + [pltpu.SemaphoreType.DMA, pltpu.SemaphoreType.DMA((2,))]
    )

    @functools.partial(
        pl.kernel,
        out_type=jax.ShapeDtypeStruct((B,), jnp.float32),
        mesh=mesh,
        scratch_types=scratch,
    )
    def k(r_idx_h, e1_h, e2_h, e3_h, e4_h, e5_h, e6_h, E_h, R_h, Rb_h,
          out_h, ri, i0, i1, i2, i3, i4, i5, rvb,
          v0, v1, v2, v3, v4, v5,
          rb_full, out_b, semi, semp):
        idx_hs = (r_idx_h, e1_h, e2_h, e3_h, e4_h, e5_h, e6_h)
        idx_vs = (ri, i0, i1, i2, i3, i4, i5)
        evbs = (v0, v1, v2, v3, v4, v5)

        wid = lax.axis_index("s") * _NC + lax.axis_index("c")
        base = wid * per_w
        lane = lax.iota(jnp.int32, _L)

        stage = [pltpu.async_copy(h.at[pl.ds(base, per_w)],
                                  iv.at[pl.ds(0, per_w)] if iv is ri else iv,
                                  semi)
                 for h, iv in zip(idx_hs, idx_vs)]
        stage.append(pltpu.async_copy(Rb_h, rb_full, semi))
        for d in stage:
            d.wait()

        def fire(sub, par):
            sem = semp.at[par]
            pltpu.async_copy(R_h.at[ri.at[pl.ds(sub * C, C)]],
                             rvb.at[par], sem)
            for a in range(A):
                pltpu.async_copy(
                    E_h.at[idx_vs[1 + a].at[pl.ds(sub * C, C)]],
                    evbs[a].at[par], sem)

        def drain(sub, par):
            sem = semp.at[par]
            pltpu.make_async_copy(R_h.at[ri.at[pl.ds(sub * C, C)]],
                                  rvb.at[par], sem).wait()
            for a in range(A):
                pltpu.make_async_copy(
                    E_h.at[idx_vs[1 + a].at[pl.ds(sub * C, C)]],
                    evbs[a].at[par], sem).wait()

        def compute(sub, par):
            rv = rvb.at[par]
            evs = tuple(e.at[par] for e in evbs)

            def group(g, carry_g):
                def item(t, acc):
                    i = g * _L + t
                    rid = ri[pl.ds(sub * C + i, _L)][0]
                    parts = [rb_full[pl.ds(rid * BQ, BQ)],
                             jnp.zeros((_L,), jnp.float32)]
                    n = 0
                    for a in range(A):
                        ev = evs[a]
                        for w in range(W):
                            parts[n % 2] = parts[n % 2] + (
                                rv[i, pl.ds(a * D + w * BQ, BQ)]
                                * ev[i, pl.ds(w * BQ, BQ)])
                            n += 1
                    inner = parts[0] + parts[1]
                    inner = jnp.maximum(inner, 0.0)
                    for step in (8, 4, 2, 1):
                        inner = inner + inner.at[lane ^ step].get(
                            mode="promise_in_bounds")
                    return jnp.where(lane == t, inner, acc)

                acc = lax.fori_loop(0, _L, item,
                                    jnp.zeros((_L,), jnp.float32))
                out_b[pl.ds(sub * C + g * _L, _L)] = acc
                return carry_g

            lax.fori_loop(0, C // _L, group, 0)

        fire(0, 0)

        def body(sub, carry):
            par = lax.rem(sub, 2)
            fire(lax.rem(sub + 1, n_sub), 1 - par)
            drain(sub, par)
            compute(sub, par)
            return carry

        lax.fori_loop(0, n_sub, body, 0)
        drain(0, n_sub % 2)

        pltpu.sync_copy(out_b, out_h.at[pl.ds(base, per_w)])

    return k


def kernel(r_idx, e1_idx, e2_idx, e3_idx, e4_idx, e5_idx, e6_idx, E, R,
           R_bias):
    B = r_idx.shape[0]
    D = E.shape[1]
    RD = R.shape[1]
    BQ = R_bias.shape[1]
    nrel = R.shape[0]
    k = _build(B, D, RD, BQ, nrel)
    to_i32 = lambda x: x.astype(jnp.int32)
    return k(to_i32(r_idx), to_i32(e1_idx), to_i32(e2_idx), to_i32(e3_idx),
             to_i32(e4_idx), to_i32(e5_idx), to_i32(e6_idx), E, R,
             R_bias.reshape(-1))


# speedup vs baseline: 5.6623x; 1.0445x over previous
import functools

import jax
import jax.numpy as jnp
from jax import lax
from jax.experimental import pallas as pl
from jax.experimental.pallas import tpu as pltpu
from jax.experimental.pallas import tpu_sc as plsc

_NC = 2
_NS = 16
_L = 16


def _build(B, D, RD, BQ, nrel):
    A = RD // D
    W = D // BQ
    NW = _NC * _NS
    per_w = B // NW
    C = 32
    n_sub = per_w // C

    mesh = plsc.VectorSubcoreMesh(core_axis_name="c", subcore_axis_name="s")

    scratch = (
        [pltpu.VMEM((per_w + _L,), jnp.int32)]
        + [pltpu.VMEM((per_w,), jnp.int32) for _ in range(A)]
        + [pltpu.VMEM((2, C, RD), jnp.float32)]
        + [pltpu.VMEM((2, C, D), jnp.float32) for _ in range(A)]
        + [pltpu.VMEM((nrel * BQ,), jnp.float32)]
        + [pltpu.VMEM((per_w,), jnp.float32)]
        + [pltpu.SemaphoreType.DMA, pltpu.SemaphoreType.DMA,
           pltpu.SemaphoreType.DMA((2,))]
    )

    @functools.partial(
        pl.kernel,
        out_type=jax.ShapeDtypeStruct((B,), jnp.float32),
        mesh=mesh,
        scratch_types=scratch,
    )
    def k(r_idx_h, e1_h, e2_h, e3_h, e4_h, e5_h, e6_h, E_h, R_h, Rb_h,
          out_h, ri, i0, i1, i2, i3, i4, i5, rvb,
          v0, v1, v2, v3, v4, v5,
          rb_full, out_b, semi, semt, semp):
        idx_hs = (r_idx_h, e1_h, e2_h, e3_h, e4_h, e5_h, e6_h)
        idx_vs = (ri, i0, i1, i2, i3, i4, i5)
        evbs = (v0, v1, v2, v3, v4, v5)

        wid = lax.axis_index("s") * _NC + lax.axis_index("c")
        base = wid * per_w
        lane = lax.iota(jnp.int32, _L)

        head = [pltpu.async_copy(h.at[pl.ds(base, C)],
                                 iv.at[pl.ds(0, C)], semi)
                for h, iv in zip(idx_hs, idx_vs)]
        rest = per_w - C
        tail = [pltpu.async_copy(h.at[pl.ds(base + C, rest)],
                                 iv.at[pl.ds(C, rest)], semt)
                for h, iv in zip(idx_hs, idx_vs)]
        tail.append(pltpu.async_copy(Rb_h, rb_full, semt))

        def fire(sub, par):
            sem = semp.at[par]
            pltpu.async_copy(R_h.at[ri.at[pl.ds(sub * C, C)]],
                             rvb.at[par], sem)
            for a in range(A):
                pltpu.async_copy(
                    E_h.at[idx_vs[1 + a].at[pl.ds(sub * C, C)]],
                    evbs[a].at[par], sem)

        def drain(sub, par):
            sem = semp.at[par]
            pltpu.make_async_copy(R_h.at[ri.at[pl.ds(sub * C, C)]],
                                  rvb.at[par], sem).wait()
            for a in range(A):
                pltpu.make_async_copy(
                    E_h.at[idx_vs[1 + a].at[pl.ds(sub * C, C)]],
                    evbs[a].at[par], sem).wait()

        def compute(sub, par):
            rv = rvb.at[par]
            evs = tuple(e.at[par] for e in evbs)

            def group(g, carry_g):
                def item(t, acc):
                    i = g * _L + t
                    rid = ri[pl.ds(sub * C + i, _L)][0]
                    parts = [rb_full[pl.ds(rid * BQ, BQ)],
                             jnp.zeros((_L,), jnp.float32)]
                    n = 0
                    for a in range(A):
                        ev = evs[a]
                        for w in range(W):
                            parts[n % 2] = parts[n % 2] + (
                                rv[i, pl.ds(a * D + w * BQ, BQ)]
                                * ev[i, pl.ds(w * BQ, BQ)])
                            n += 1
                    inner = parts[0] + parts[1]
                    inner = jnp.maximum(inner, 0.0)
                    for step in (8, 4, 2, 1):
                        inner = inner + inner.at[lane ^ step].get(
                            mode="promise_in_bounds")
                    return jnp.where(lane == t, inner, acc)

                acc = lax.fori_loop(0, _L, item,
                                    jnp.zeros((_L,), jnp.float32))
                out_b[pl.ds(sub * C + g * _L, _L)] = acc
                return carry_g

            lax.fori_loop(0, C // _L, group, 0)

        for d in head:
            d.wait()
        fire(0, 0)
        for d in tail:
            d.wait()

        def body(sub, carry):
            par = lax.rem(sub, 2)
            fire(lax.rem(sub + 1, n_sub), 1 - par)
            drain(sub, par)
            compute(sub, par)
            return carry

        lax.fori_loop(0, n_sub, body, 0)
        drain(0, n_sub % 2)

        pltpu.sync_copy(out_b, out_h.at[pl.ds(base, per_w)])

    return k


def kernel(r_idx, e1_idx, e2_idx, e3_idx, e4_idx, e5_idx, e6_idx, E, R,
           R_bias):
    B = r_idx.shape[0]
    D = E.shape[1]
    RD = R.shape[1]
    BQ = R_bias.shape[1]
    nrel = R.shape[0]
    k = _build(B, D, RD, BQ, nrel)
    to_i32 = lambda x: x.astype(jnp.int32)
    return k(to_i32(r_idx), to_i32(e1_idx), to_i32(e2_idx), to_i32(e3_idx),
             to_i32(e4_idx), to_i32(e5_idx), to_i32(e6_idx), E, R,
             R_bias.reshape(-1))
